# baseline (speedup 1.0000x reference)
import functools

import jax
import jax.numpy as jnp
from jax import lax
from jax.experimental import pallas as pl
from jax.experimental.pallas import tpu as pltpu
from jax.experimental.pallas import tpu_sc as plsc

NC = 2
NS = 16
NW = NC * NS
LANES = 16

STEP = 256
SUB = 128
TCHUNK = 1024

_SC_PARAMS_LINEAR = pltpu.CompilerParams(
    needs_layout_passes=False, use_tc_tiling_on_sc=False
)
_SC_PARAMS_TILED = pltpu.CompilerParams(
    needs_layout_passes=False, use_tc_tiling_on_sc=True
)


def _mesh():
    return plsc.VectorSubcoreMesh(
        core_axis_name="c", subcore_axis_name="s", num_cores=NC, num_subcores=NS
    )


def _make_transpose_kernel(C, P):
    n_full = P // TCHUNK
    tail = P - n_full * TCHUNK
    tail_rows = tail * C // 128
    out_rows = P * C // 128
    rpc = TCHUNK * C // 128
    n_groups = TCHUNK // LANES

    @functools.partial(
        pl.kernel,
        out_type=jax.ShapeDtypeStruct((out_rows, 128), jnp.float32),
        mesh=_mesh(),
        compiler_params=_SC_PARAMS_TILED,
        scratch_types=[
            pltpu.VMEM((2, C, TCHUNK), jnp.float32),
            pltpu.VMEM((2, rpc, 128), jnp.float32),
            pltpu.SemaphoreType.DMA,
            pltpu.SemaphoreType.DMA,
            pltpu.SemaphoreType.DMA,
            pltpu.SemaphoreType.DMA,
        ],
    )
    def tr_kernel(src_hbm, tail_hbm, out_hbm, in_v, out_v,
                  sem_in0, sem_in1, sem_out0, sem_out1):
        cid = lax.axis_index("c")
        sid = lax.axis_index("s")
        wid = sid * NC + cid
        sem_in = [sem_in0, sem_in1]
        sem_out = [sem_out0, sem_out1]

        iota16 = lax.iota(jnp.int32, LANES)
        rdiv = lax.shift_right_logical(iota16, 3)
        colbase = (iota16 & 7) * LANES

        n_my = (n_full - wid + NW - 1) // NW

        def fire_in(i, b):
            @pl.when(i < n_my)
            def _():
                p0 = (i * NW + wid) * TCHUNK
                pltpu.async_copy(
                    src_hbm.at[:, pl.ds(p0, TCHUNK)], in_v.at[b], sem_in[b]
                )

        def wait_in(b):
            pltpu.make_async_copy(
                src_hbm.at[:, pl.ds(0, TCHUNK)], in_v.at[b], sem_in[b]
            ).wait()

        def compute(b):
            def grp(g, c2):
                rowv = rdiv + 2 * g
                for c in range(C):
                    xv = in_v[b, c, pl.ds(g * LANES, LANES)]
                    plsc.store_scatter(
                        out_v.at[b], [rowv, colbase + c], xv
                    )
                return c2
            lax.fori_loop(0, n_groups, grp, 0)

        def fire_out(i, b):
            r0 = (i * NW + wid) * rpc
            pltpu.async_copy(
                out_v.at[b], out_hbm.at[pl.ds(r0, rpc), :], sem_out[b]
            )

        def wait_out(b):
            pltpu.make_async_copy(
                out_v.at[b], out_hbm.at[pl.ds(0, rpc), :], sem_out[b]
            ).wait()

        if tail:
            @pl.when(wid == NW - 1)
            def _():
                pltpu.sync_copy(tail_hbm, out_v.at[0, pl.ds(0, tail_rows), :])
                pltpu.sync_copy(
                    out_v.at[0, pl.ds(0, tail_rows), :],
                    out_hbm.at[pl.ds(n_full * rpc, tail_rows), :],
                )

        fire_in(0, 0)
        fire_in(1, 1)

        def pair(i2, c2):
            for b in range(2):
                i = i2 * 2 + b

                @pl.when(i < n_my)
                def _():
                    wait_in(b)
                    @pl.when(i >= 2)
                    def _():
                        wait_out(b)
                    compute(b)
                    fire_out(i, b)
                    fire_in(i + 2, b)
            return c2

        lax.fori_loop(0, (n_my + 1) // 2 + 1, pair, 0)

        wait_out(0)
        wait_out(1)

    return tr_kernel


def _build_table(ptclds):
    C, P = ptclds.shape
    n_full = P // TCHUNK
    tail = P - n_full * TCHUNK
    tail_hbm = (
        ptclds[:, n_full * TCHUNK:].T.reshape(tail * C // 128, 128)
        if tail
        else jnp.zeros((1, 128), jnp.float32)
    )
    out2 = _make_transpose_kernel(C, P)(ptclds, tail_hbm)
    return out2.reshape(P, C)


def _make_sc_kernel(N, K, HW, C, P):
    n_pix = N * HW
    pix_per_tile = n_pix // NW
    n_steps = pix_per_tile // STEP
    tiles_per_img = HW // pix_per_tile
    n_sub = STEP // SUB

    @functools.partial(
        pl.kernel,
        out_type=jax.ShapeDtypeStruct((N * C, HW), jnp.float32),
        mesh=_mesh(),
        compiler_params=_SC_PARAMS_LINEAR,
        scratch_types=[
            pltpu.VMEM((3, K, STEP), jnp.int32),
            pltpu.VMEM((3, K, STEP), jnp.float32),
            pltpu.VMEM((3, K * STEP, C), jnp.float32),
            pltpu.VMEM((3, C, STEP), jnp.float32),
            pltpu.SemaphoreType.DMA,
            pltpu.SemaphoreType.DMA,
            pltpu.SemaphoreType.DMA,
            pltpu.SemaphoreType.DMA,
            pltpu.SemaphoreType.DMA,
            pltpu.SemaphoreType.DMA,
            pltpu.SemaphoreType.DMA,
            pltpu.SemaphoreType.DMA,
            pltpu.SemaphoreType.DMA,
        ],
    )
    def sc_kernel(frag_hbm, alpha_hbm, table_hbm, out_hbm,
                  idx_v, alpha_v, rows_v, out_stage,
                  sem_in0, sem_in1, sem_in2, sem_g0, sem_g1, sem_g2,
                  sem_out0, sem_out1, sem_out2):
        cid = lax.axis_index("c")
        sid = lax.axis_index("s")
        wid = sid * NC + cid
        n = wid // tiles_per_img
        col0 = (wid % tiles_per_img) * pix_per_tile
        sem_in = [sem_in0, sem_in1, sem_in2]
        sem_g = [sem_g0, sem_g1, sem_g2]
        sem_out = [sem_out0, sem_out1, sem_out2]

        iota16 = lax.iota(jnp.int32, LANES)

        def col_of(s):
            return col0 + s * STEP

        def fire_in(s, b):
            @pl.when(s < n_steps)
            def _():
                col = col_of(s)
                pltpu.async_copy(
                    frag_hbm.at[n, :, pl.ds(col, STEP)], idx_v.at[b], sem_in[b]
                )
                pltpu.async_copy(
                    alpha_hbm.at[n, :, pl.ds(col, STEP)], alpha_v.at[b],
                    sem_in[b],
                )

        def wait_in(b):
            pltpu.make_async_copy(
                frag_hbm.at[0, :, pl.ds(0, STEP)], idx_v.at[b], sem_in[b]
            ).wait()
            pltpu.make_async_copy(
                alpha_hbm.at[0, :, pl.ds(0, STEP)], alpha_v.at[b], sem_in[b]
            ).wait()

        def fire_gathers(j, b):
            for k in range(K):
                for hf in range(n_sub):
                    pltpu.async_copy(
                        table_hbm.at[idx_v.at[j, k, pl.ds(hf * SUB, SUB)]],
                        rows_v.at[b, pl.ds(k * STEP + hf * SUB, SUB), :],
                        sem_g[b],
                    )

        def wait_gathers(j, b):
            for k in range(K):
                for hf in range(n_sub):
                    pltpu.make_async_copy(
                        table_hbm.at[idx_v.at[j, k, pl.ds(hf * SUB, SUB)]],
                        rows_v.at[b, pl.ds(k * STEP + hf * SUB, SUB), :],
                        sem_g[b],
                    ).wait()

        def compute(j, b):
            def group(g, c2):
                gsl = pl.ds(g * LANES, LANES)
                a = [alpha_v[j, k, gsl] for k in range(K)]
                d = a[0]
                for k in range(1, K):
                    d = d + a[k]
                r = 1.0 / jnp.maximum(d, 1e-10)
                w = [ak * r for ak in a]
                pvec = g * LANES + iota16
                rowvecs = [pvec + k * STEP for k in range(K)]
                cvecs = [jnp.full((LANES,), c, jnp.int32) for c in range(C)]
                for c in range(C):
                    acc = w[0] * plsc.load_gather(
                        rows_v.at[b], [rowvecs[0], cvecs[c]])
                    for k in range(1, K):
                        acc = acc + w[k] * plsc.load_gather(
                            rows_v.at[b], [rowvecs[k], cvecs[c]])
                    out_stage[b, c, gsl] = acc
                return c2

            lax.fori_loop(0, STEP // LANES, group, 0)

        def fire_out(s, b):
            pltpu.async_copy(
                out_stage.at[b],
                out_hbm.at[pl.ds(n * C, C), pl.ds(col_of(s), STEP)],
                sem_out[b],
            )

        def wait_out(b):
            pltpu.make_async_copy(
                out_stage.at[b],
                out_hbm.at[pl.ds(0, C), pl.ds(0, STEP)],
                sem_out[b],
            ).wait()

        fire_in(0, 0)
        fire_in(1, 1)
        wait_in(0)
        fire_gathers(0, 0)

        def tri(s3, c2):
            for u in range(3):
                s = s3 * 3 + u
                jn = (u + 1) % 3
                jp = (u + 2) % 3

                @pl.when(s + 1 < n_steps)
                def _():
                    wait_in(jn)
                    fire_gathers(jn, jn)

                @pl.when((s >= 1) & (s <= n_steps))
                def _():
                    wait_gathers(jp, jp)
                    @pl.when(s >= 4)
                    def _():
                        wait_out(jp)
                    compute(jp, jp)
                    fire_out(s - 1, jp)

                @pl.when(s + 2 < n_steps)
                def _():
                    fire_in(s + 2, jp)
            return c2

        lax.fori_loop(0, (n_steps + 1 + 2) // 3, tri, 0)

        wait_out(n_steps % 3)
        wait_out((n_steps + 1) % 3)
        wait_out((n_steps + 2) % 3)

    return sc_kernel


def kernel(fragments, alphas, ptclds):
    N, K, H, W = fragments.shape
    C, P = ptclds.shape
    HW = H * W

    table = _build_table(ptclds)
    frag = fragments.reshape(N, K, HW).astype(jnp.int32)
    alph = alphas.reshape(N, K, HW)

    sc_kernel = _make_sc_kernel(N, K, HW, C, P)
    out = sc_kernel(frag, alph, table)
    return out.reshape(N, C, H, W)


# speedup vs baseline: 1.9967x; 1.9967x over previous
import functools

import jax
import jax.numpy as jnp
from jax import lax
from jax.experimental import pallas as pl
from jax.experimental.pallas import tpu as pltpu
from jax.experimental.pallas import tpu_sc as plsc

NC = 2
NS = 16
NW = NC * NS
LANES = 16

STEP = 256
SUB = 128
TCHUNK = 1024

_SC_PARAMS_LINEAR = pltpu.CompilerParams(
    needs_layout_passes=False, use_tc_tiling_on_sc=False
)
_SC_PARAMS_TILED = pltpu.CompilerParams(
    needs_layout_passes=False, use_tc_tiling_on_sc=True
)


def _mesh():
    return plsc.VectorSubcoreMesh(
        core_axis_name="c", subcore_axis_name="s", num_cores=NC, num_subcores=NS
    )


def _make_transpose_kernel(C, P):
    n_full = P // TCHUNK
    tail = P - n_full * TCHUNK
    cp = C // 2
    tail_rows = tail * cp // 128
    out_rows = P * cp // 128
    rpc = TCHUNK * cp // 128
    n_groups = TCHUNK // LANES

    @functools.partial(
        pl.kernel,
        out_type=jax.ShapeDtypeStruct((out_rows, 128), jnp.float32),
        mesh=_mesh(),
        compiler_params=_SC_PARAMS_TILED,
        scratch_types=[
            pltpu.VMEM((2, C, TCHUNK), jnp.float32),
            pltpu.VMEM((2, rpc, 128), jnp.float32),
            pltpu.SemaphoreType.DMA,
            pltpu.SemaphoreType.DMA,
            pltpu.SemaphoreType.DMA,
            pltpu.SemaphoreType.DMA,
        ],
    )
    def tr_kernel(src_hbm, tail_hbm, out_hbm, in_v, out_v,
                  sem_in0, sem_in1, sem_out0, sem_out1):
        cid = lax.axis_index("c")
        sid = lax.axis_index("s")
        wid = sid * NC + cid
        sem_in = [sem_in0, sem_in1]
        sem_out = [sem_out0, sem_out1]

        iota16 = lax.iota(jnp.int32, LANES)
        colbase = iota16 * cp

        n_my = (n_full - wid + NW - 1) // NW

        def fire_in(i, b):
            @pl.when(i < n_my)
            def _():
                p0 = (i * NW + wid) * TCHUNK
                pltpu.async_copy(
                    src_hbm.at[:, pl.ds(p0, TCHUNK)], in_v.at[b], sem_in[b]
                )

        def wait_in(b):
            pltpu.make_async_copy(
                src_hbm.at[:, pl.ds(0, TCHUNK)], in_v.at[b], sem_in[b]
            ).wait()

        def compute(b):
            def grp(g, c2):
                rowv = jnp.full((LANES,), g, jnp.int32)
                for j in range(cp):
                    a0 = in_v[b, 2 * j, pl.ds(g * LANES, LANES)]
                    a1 = in_v[b, 2 * j + 1, pl.ds(g * LANES, LANES)]
                    pk = plsc.bitcast(
                        plsc.pack(a0, a1, format=plsc.PackFormat.INTERLEAVED),
                        jnp.float32,
                    )
                    plsc.store_scatter(out_v.at[b], [rowv, colbase + j], pk)
                return c2
            lax.fori_loop(0, n_groups, grp, 0)

        def fire_out(i, b):
            r0 = (i * NW + wid) * rpc
            pltpu.async_copy(
                out_v.at[b], out_hbm.at[pl.ds(r0, rpc), :], sem_out[b]
            )

        def wait_out(b):
            pltpu.make_async_copy(
                out_v.at[b], out_hbm.at[pl.ds(0, rpc), :], sem_out[b]
            ).wait()

        if tail:
            @pl.when(wid == NW - 1)
            def _():
                pltpu.sync_copy(tail_hbm, out_v.at[0, pl.ds(0, tail_rows), :])
                pltpu.sync_copy(
                    out_v.at[0, pl.ds(0, tail_rows), :],
                    out_hbm.at[pl.ds(n_full * rpc, tail_rows), :],
                )

        fire_in(0, 0)
        fire_in(1, 1)

        def pair(i2, c2):
            for b in range(2):
                i = i2 * 2 + b

                @pl.when(i < n_my)
                def _():
                    wait_in(b)
                    @pl.when(i >= 2)
                    def _():
                        wait_out(b)
                    compute(b)
                    fire_out(i, b)
                    fire_in(i + 2, b)
            return c2

        lax.fori_loop(0, (n_my + 1) // 2 + 1, pair, 0)

        wait_out(0)
        wait_out(1)

    return tr_kernel


def _build_table(ptclds):
    C, P = ptclds.shape
    n_full = P // TCHUNK
    tail = P - n_full * TCHUNK
    if tail:
        t = ptclds[:, n_full * TCHUNK:].T.astype(jnp.bfloat16)
        tw = jax.lax.bitcast_convert_type(
            t.reshape(tail, C // 2, 2), jnp.float32
        )
        tail_hbm = tw.reshape(tail * C // 2 // 128, 128)
    else:
        tail_hbm = jnp.zeros((1, 128), jnp.float32)
    out2 = _make_transpose_kernel(C, P)(ptclds, tail_hbm)
    return out2.reshape(P, C // 2)


def _make_sc_kernel(N, K, HW, C, P):
    n_pix = N * HW
    pix_per_tile = n_pix // NW
    n_steps = pix_per_tile // STEP
    tiles_per_img = HW // pix_per_tile
    n_sub = STEP // SUB

    @functools.partial(
        pl.kernel,
        out_type=jax.ShapeDtypeStruct((N * C, HW), jnp.float32),
        mesh=_mesh(),
        compiler_params=_SC_PARAMS_LINEAR,
        scratch_types=[
            pltpu.VMEM((3, K, STEP), jnp.int32),
            pltpu.VMEM((3, K, STEP), jnp.float32),
            pltpu.VMEM((3, K * STEP, C // 2), jnp.float32),
            pltpu.VMEM((3, C, STEP), jnp.float32),
            pltpu.SemaphoreType.DMA,
            pltpu.SemaphoreType.DMA,
            pltpu.SemaphoreType.DMA,
            pltpu.SemaphoreType.DMA,
            pltpu.SemaphoreType.DMA,
            pltpu.SemaphoreType.DMA,
            pltpu.SemaphoreType.DMA,
            pltpu.SemaphoreType.DMA,
            pltpu.SemaphoreType.DMA,
        ],
    )
    def sc_kernel(frag_hbm, alpha_hbm, table_hbm, out_hbm,
                  idx_v, alpha_v, rows_v, out_stage,
                  sem_in0, sem_in1, sem_in2, sem_g0, sem_g1, sem_g2,
                  sem_out0, sem_out1, sem_out2):
        cid = lax.axis_index("c")
        sid = lax.axis_index("s")
        wid = sid * NC + cid
        n = wid // tiles_per_img
        col0 = (wid % tiles_per_img) * pix_per_tile
        sem_in = [sem_in0, sem_in1, sem_in2]
        sem_g = [sem_g0, sem_g1, sem_g2]
        sem_out = [sem_out0, sem_out1, sem_out2]

        iota16 = lax.iota(jnp.int32, LANES)

        def col_of(s):
            return col0 + s * STEP

        def fire_in(s, b):
            @pl.when(s < n_steps)
            def _():
                col = col_of(s)
                pltpu.async_copy(
                    frag_hbm.at[n, :, pl.ds(col, STEP)], idx_v.at[b], sem_in[b]
                )
                pltpu.async_copy(
                    alpha_hbm.at[n, :, pl.ds(col, STEP)], alpha_v.at[b],
                    sem_in[b],
                )

        def wait_in(b):
            pltpu.make_async_copy(
                frag_hbm.at[0, :, pl.ds(0, STEP)], idx_v.at[b], sem_in[b]
            ).wait()
            pltpu.make_async_copy(
                alpha_hbm.at[0, :, pl.ds(0, STEP)], alpha_v.at[b], sem_in[b]
            ).wait()

        def fire_gathers(j, b):
            for k in range(K):
                for hf in range(n_sub):
                    pltpu.async_copy(
                        table_hbm.at[idx_v.at[j, k, pl.ds(hf * SUB, SUB)]],
                        rows_v.at[b, pl.ds(k * STEP + hf * SUB, SUB), :],
                        sem_g[b],
                    )

        def wait_gathers(j, b):
            for k in range(K):
                for hf in range(n_sub):
                    pltpu.make_async_copy(
                        table_hbm.at[idx_v.at[j, k, pl.ds(hf * SUB, SUB)]],
                        rows_v.at[b, pl.ds(k * STEP + hf * SUB, SUB), :],
                        sem_g[b],
                    ).wait()

        def compute(j, b):
            def group(g, c2):
                gsl = pl.ds(g * LANES, LANES)
                a = [alpha_v[j, k, gsl] for k in range(K)]
                d = a[0]
                for k in range(1, K):
                    d = d + a[k]
                r = 1.0 / jnp.maximum(d, 1e-10)
                w = [ak * r for ak in a]
                pvec = g * LANES + iota16
                rowvecs = [pvec + k * STEP for k in range(K)]
                cvecs = [jnp.full((LANES,), jj, jnp.int32)
                         for jj in range(C // 2)]
                for jj in range(C // 2):
                    acc0 = None
                    acc1 = None
                    for k in range(K):
                        pk = plsc.load_gather(
                            rows_v.at[b], [rowvecs[k], cvecs[jj]])
                        u0, u1 = plsc.unpack(
                            plsc.bitcast(pk, jnp.bfloat16),
                            format=plsc.PackFormat.INTERLEAVED,
                            preferred_element_type=jnp.float32,
                        )
                        if acc0 is None:
                            acc0 = w[k] * u0
                            acc1 = w[k] * u1
                        else:
                            acc0 = acc0 + w[k] * u0
                            acc1 = acc1 + w[k] * u1
                    out_stage[b, 2 * jj, gsl] = acc0
                    out_stage[b, 2 * jj + 1, gsl] = acc1
                return c2

            lax.fori_loop(0, STEP // LANES, group, 0)

        def fire_out(s, b):
            pltpu.async_copy(
                out_stage.at[b],
                out_hbm.at[pl.ds(n * C, C), pl.ds(col_of(s), STEP)],
                sem_out[b],
            )

        def wait_out(b):
            pltpu.make_async_copy(
                out_stage.at[b],
                out_hbm.at[pl.ds(0, C), pl.ds(0, STEP)],
                sem_out[b],
            ).wait()

        fire_in(0, 0)
        fire_in(1, 1)
        wait_in(0)
        fire_gathers(0, 0)

        def tri(s3, c2):
            for u in range(3):
                s = s3 * 3 + u
                jn = (u + 1) % 3
                jp = (u + 2) % 3

                @pl.when(s + 1 < n_steps)
                def _():
                    wait_in(jn)
                    fire_gathers(jn, jn)

                @pl.when((s >= 1) & (s <= n_steps))
                def _():
                    wait_gathers(jp, jp)
                    @pl.when(s >= 4)
                    def _():
                        wait_out(jp)
                    compute(jp, jp)
                    fire_out(s - 1, jp)

                @pl.when(s + 2 < n_steps)
                def _():
                    fire_in(s + 2, jp)
            return c2

        lax.fori_loop(0, (n_steps + 1 + 2) // 3, tri, 0)

        wait_out(n_steps % 3)
        wait_out((n_steps + 1) % 3)
        wait_out((n_steps + 2) % 3)

    return sc_kernel


def kernel(fragments, alphas, ptclds):
    N, K, H, W = fragments.shape
    C, P = ptclds.shape
    HW = H * W

    table = _build_table(ptclds)
    frag = fragments.reshape(N, K, HW).astype(jnp.int32)
    alph = alphas.reshape(N, K, HW)

    sc_kernel = _make_sc_kernel(N, K, HW, C, P)
    out = sc_kernel(frag, alph, table)
    return out.reshape(N, C, H, W)
